# Initial kernel scaffold; baseline (speedup 1.0000x reference)
import jax
import jax.numpy as jnp
from jax.experimental import pallas as pl


def kernel(x, coords, conv1, bn1_g, bn1_b, bn1_m, bn1_v, l0_b0_conv1, l0_b0_bn1_g, l0_b0_bn1_b, l0_b0_bn1_m, l0_b0_bn1_v, l0_b0_conv2, l0_b0_bn2_g, l0_b0_bn2_b, l0_b0_bn2_m, l0_b0_bn2_v, l0_b1_conv1, l0_b1_bn1_g, l0_b1_bn1_b, l0_b1_bn1_m, l0_b1_bn1_v, l0_b1_conv2, l0_b1_bn2_g, l0_b1_bn2_b, l0_b1_bn2_m, l0_b1_bn2_v, l1_b0_conv1, l1_b0_bn1_g, l1_b0_bn1_b, l1_b0_bn1_m, l1_b0_bn1_v, l1_b0_conv2, l1_b0_bn2_g, l1_b0_bn2_b, l1_b0_bn2_m, l1_b0_bn2_v, l1_b0_dconv, l1_b0_dbn_g, l1_b0_dbn_b, l1_b0_dbn_m, l1_b0_dbn_v, l1_b1_conv1, l1_b1_bn1_g, l1_b1_bn1_b, l1_b1_bn1_m, l1_b1_bn1_v, l1_b1_conv2, l1_b1_bn2_g, l1_b1_bn2_b, l1_b1_bn2_m, l1_b1_bn2_v, l2_b0_conv1, l2_b0_bn1_g, l2_b0_bn1_b, l2_b0_bn1_m, l2_b0_bn1_v, l2_b0_conv2, l2_b0_bn2_g, l2_b0_bn2_b, l2_b0_bn2_m, l2_b0_bn2_v, l2_b0_dconv, l2_b0_dbn_g, l2_b0_dbn_b, l2_b0_dbn_m, l2_b0_dbn_v, l2_b1_conv1, l2_b1_bn1_g, l2_b1_bn1_b, l2_b1_bn1_m, l2_b1_bn1_v, l2_b1_conv2, l2_b1_bn2_g, l2_b1_bn2_b, l2_b1_bn2_m, l2_b1_bn2_v, l3_b0_conv1, l3_b0_bn1_g, l3_b0_bn1_b, l3_b0_bn1_m, l3_b0_bn1_v, l3_b0_conv2, l3_b0_bn2_g, l3_b0_bn2_b, l3_b0_bn2_m, l3_b0_bn2_v, l3_b0_dconv, l3_b0_dbn_g, l3_b0_dbn_b, l3_b0_dbn_m, l3_b0_dbn_v, l3_b1_conv1, l3_b1_bn1_g, l3_b1_bn1_b, l3_b1_bn1_m, l3_b1_bn1_v, l3_b1_conv2, l3_b1_bn2_g, l3_b1_bn2_b, l3_b1_bn2_m, l3_b1_bn2_v, fc1_w, fc1_b, dec1_w, dec1_b, dec2_w, dec2_b):
    raise NotImplementedError("write your pallas kernel here")


# speedup vs baseline: 14.0454x; 14.0454x over previous
import functools

import jax
import jax.numpy as jnp
from jax.experimental import pallas as pl
from jax.experimental.pallas import tpu as pltpu

_VMEM_LIMIT = 48 * 1024 * 1024

_TAP3 = {0: (0, 1), 1: (1, 0), 2: (1, 1)}
_TAP7 = {0: (0, 1), 1: (1, 0), 2: (1, 1), 3: (2, 0), 4: (2, 1), 5: (3, 0),
         6: (3, 1)}


def _fold_bn(g, b, m, v):
    s = g / jnp.sqrt(v + 1e-5)
    return (s.reshape(1, -1).astype(jnp.float32),
            (b - m * s).reshape(1, -1).astype(jnp.float32))


def _w_s1(w):
    return jnp.transpose(w, (2, 3, 1, 0)).reshape(
        9 * w.shape[1], w.shape[0]).astype(jnp.bfloat16)


def _w_s2(w, taps, mapping):
    o, c, k, _ = w.shape
    w2 = jnp.zeros((taps, taps, 2, 2, c, o), w.dtype)
    for dy in range(k):
        ty, py = mapping[dy]
        for dx in range(k):
            tx, px = mapping[dx]
            w2 = w2.at[ty, tx, py, px].set(jnp.transpose(w[:, :, dy, dx]))
    return w2.reshape(taps * taps * 4 * c, o).astype(jnp.bfloat16)


def _s2d(x):
    n, h, w, c = x.shape
    x = x.reshape(n, h // 2, 2, w // 2, 2, c)
    x = jnp.transpose(x, (0, 1, 3, 2, 4, 5))
    return x.reshape(n, h // 2, w // 2, 4 * c)


def _cols(x, t, pt, pb):
    h, w, c = x.shape
    ho = h + pt + pb - t + 1
    xp = jnp.pad(x, ((pt, pb), (pt, pb), (0, 0)))
    cat = jnp.concatenate([xp[:, dx:dx + ho, :] for dx in range(t)], axis=2)
    cols = jnp.concatenate([cat[dy:dy + ho, :, :] for dy in range(t)], axis=2)
    return cols.reshape(ho * ho, t * t * c)


def _maxpool_3x3_s2_p1(y):
    h, w, c = y.shape
    yr = y.reshape(h // 2, 2, w, c)
    m0, m1 = yr[:, 0], yr[:, 1]
    m1p = jnp.concatenate([jnp.zeros_like(m1[:1]), m1[:-1]], axis=0)
    rm = jnp.maximum(jnp.maximum(m0, m1), m1p)
    rr = rm.reshape(h // 2, w // 2, 2, c)
    c0, c1 = rr[:, :, 0], rr[:, :, 1]
    c1p = jnp.concatenate([jnp.zeros_like(c1[:, :1]), c1[:, :-1]], axis=1)
    return jnp.maximum(jnp.maximum(c0, c1), c1p)


def _conv_s1(inp, w_ref, s_ref, b_ref, res=None):
    h, w, _ = inp.shape
    acc = jnp.dot(_cols(inp, 3, 1, 1), w_ref[...],
                  preferred_element_type=jnp.float32)
    y = acc * s_ref[...] + b_ref[...]
    if res is not None:
        y = y + res.reshape(h * w, -1).astype(jnp.float32)
    return jnp.maximum(y, 0.0).astype(jnp.bfloat16)


def _stem_kernel(x_ref, w_ref, s_ref, b_ref, o_ref):
    x = x_ref[0]
    h = x.shape[0]
    cols = _cols(x, 4, 2, 1)
    acc = jnp.dot(cols, w_ref[...], preferred_element_type=jnp.float32)
    y = jnp.maximum(acc * s_ref[...] + b_ref[...], 0.0).astype(jnp.bfloat16)
    o_ref[0] = _maxpool_3x3_s2_p1(y.reshape(h, h, -1))


def _layer1_kernel(x_ref, w1_ref, s1_ref, b1_ref, w2_ref, s2_ref, b2_ref,
                   w3_ref, s3_ref, b3_ref, w4_ref, s4_ref, b4_ref, o_ref):
    x = x_ref[0]
    h, w, c = x.shape
    t = _conv_s1(x, w1_ref, s1_ref, b1_ref).reshape(h, w, c)
    b0 = _conv_s1(t, w2_ref, s2_ref, b2_ref, res=x).reshape(h, w, c)
    t = _conv_s1(b0, w3_ref, s3_ref, b3_ref).reshape(h, w, c)
    o_ref[0] = _conv_s1(t, w4_ref, s4_ref, b4_ref, res=b0).reshape(h, w, c)


def _layer_body(x_ref, wc1_ref, sc1_ref, bc1_ref, wd_ref, sd_ref, bd_ref,
                wc2_ref, sc2_ref, bc2_ref, wc3_ref, sc3_ref, bc3_ref,
                wc4_ref, sc4_ref, bc4_ref, cin):
    x = x_ref[0]
    h, w = x.shape[0], x.shape[1]
    o = wd_ref.shape[1]
    acc = jnp.dot(_cols(x, 2, 1, 0), wc1_ref[...],
                  preferred_element_type=jnp.float32)
    t = jnp.maximum(acc * sc1_ref[...] + bc1_ref[...],
                    0.0).astype(jnp.bfloat16).reshape(h, w, o)
    iden = jnp.dot(x[:, :, :cin].reshape(h * w, cin), wd_ref[...],
                   preferred_element_type=jnp.float32)
    iden = (iden * sd_ref[...] + bd_ref[...]).astype(jnp.bfloat16)
    b0 = _conv_s1(t, wc2_ref, sc2_ref, bc2_ref, res=iden)
    t2 = _conv_s1(b0.reshape(h, w, o), wc3_ref, sc3_ref, bc3_ref)
    b1 = _conv_s1(t2.reshape(h, w, o), wc4_ref, sc4_ref, bc4_ref, res=b0)
    return b1, h, w, o


def _layer_kernel(x_ref, wc1_ref, sc1_ref, bc1_ref, wd_ref, sd_ref, bd_ref,
                  wc2_ref, sc2_ref, bc2_ref, wc3_ref, sc3_ref, bc3_ref,
                  wc4_ref, sc4_ref, bc4_ref, o_ref, *, cin):
    b1, h, w, o = _layer_body(
        x_ref, wc1_ref, sc1_ref, bc1_ref, wd_ref, sd_ref, bd_ref,
        wc2_ref, sc2_ref, bc2_ref, wc3_ref, sc3_ref, bc3_ref,
        wc4_ref, sc4_ref, bc4_ref, cin)
    o_ref[0] = b1.reshape(h, w, o)


def _layer4_kernel(x_ref, wc1_ref, sc1_ref, bc1_ref, wd_ref, sd_ref, bd_ref,
                   wc2_ref, sc2_ref, bc2_ref, wc3_ref, sc3_ref, bc3_ref,
                   wc4_ref, sc4_ref, bc4_ref, fc1w_ref, fc1b_ref,
                   w1f_ref, d1b_ref, o_ref, *, cin):
    b1, _, _, _ = _layer_body(
        x_ref, wc1_ref, sc1_ref, bc1_ref, wd_ref, sd_ref, bd_ref,
        wc2_ref, sc2_ref, bc2_ref, wc3_ref, sc3_ref, bc3_ref,
        wc4_ref, sc4_ref, bc4_ref, cin)
    feat = jnp.mean(b1.astype(jnp.float32), axis=0, keepdims=True)
    f1 = jnp.dot(feat.astype(jnp.bfloat16), fc1w_ref[...],
                 preferred_element_type=jnp.float32) + fc1b_ref[...]
    fp = jnp.dot(f1.astype(jnp.bfloat16), w1f_ref[...],
                 preferred_element_type=jnp.float32) + d1b_ref[...]
    o_ref[0] = fp


def _dec_kernel(fp_ref, uv_ref, wxy_ref, w2_ref, b2_ref, o_ref):
    base = fp_ref[0]
    uv = uv_ref[0]
    wxy = wxy_ref[...]
    hd = base + wxy[:, 0:1] * uv[0:1, :] + wxy[:, 1:2] * uv[1:2, :]
    hd = jnp.maximum(hd, 0.0)
    y = jnp.dot(w2_ref[...], hd, preferred_element_type=jnp.float32)
    o_ref[0] = jnp.tanh(y + b2_ref[...])


def _const2(shape):
    return pl.BlockSpec(shape, lambda b: (0, 0))


def _params():
    return pltpu.CompilerParams(dimension_semantics=("parallel",),
                                vmem_limit_bytes=_VMEM_LIMIT)


def _stem(xs, w, s, b):
    n, h, _, c = xs.shape
    ho = h // 2
    return pl.pallas_call(
        _stem_kernel,
        out_shape=jax.ShapeDtypeStruct((n, ho, ho, 64), jnp.bfloat16),
        grid=(n,),
        in_specs=[
            pl.BlockSpec((1, h, h, c), lambda i: (i, 0, 0, 0)),
            _const2(w.shape), _const2(s.shape), _const2(b.shape),
        ],
        out_specs=pl.BlockSpec((1, ho, ho, 64), lambda i: (i, 0, 0, 0)),
        compiler_params=_params(),
    )(xs, w, s, b)


def _layer1(x, wsb):
    n, h, _, c = x.shape
    args = [x]
    specs = [pl.BlockSpec((1, h, h, c), lambda i: (i, 0, 0, 0))]
    for w, s, b in wsb:
        args += [w, s, b]
        specs += [_const2(w.shape), _const2(s.shape), _const2(b.shape)]
    return pl.pallas_call(
        _layer1_kernel,
        out_shape=jax.ShapeDtypeStruct((n, h, h, c), jnp.bfloat16),
        grid=(n,),
        in_specs=specs,
        out_specs=pl.BlockSpec((1, h, h, c), lambda i: (i, 0, 0, 0)),
        compiler_params=_params(),
    )(*args)


def _layer(xs, wsb, cin, cout, head=None):
    n, h, _, c4 = xs.shape
    args = [xs]
    specs = [pl.BlockSpec((1, h, h, c4), lambda i: (i, 0, 0, 0))]
    for w, s, b in wsb:
        args += [w, s, b]
        specs += [_const2(w.shape), _const2(s.shape), _const2(b.shape)]
    if head is None:
        return pl.pallas_call(
            functools.partial(_layer_kernel, cin=cin),
            out_shape=jax.ShapeDtypeStruct((n, h, h, cout), jnp.bfloat16),
            grid=(n,),
            in_specs=specs,
            out_specs=pl.BlockSpec((1, h, h, cout), lambda i: (i, 0, 0, 0)),
            compiler_params=_params(),
        )(*args)
    fc1w, fc1b, w1f, d1b = head
    args += [fc1w, fc1b, w1f, d1b]
    specs += [_const2(fc1w.shape), _const2(fc1b.shape),
              _const2(w1f.shape), _const2(d1b.shape)]
    h1 = d1b.shape[1]
    return pl.pallas_call(
        functools.partial(_layer4_kernel, cin=cin),
        out_shape=jax.ShapeDtypeStruct((n, 1, h1), jnp.float32),
        grid=(n,),
        in_specs=specs,
        out_specs=pl.BlockSpec((1, 1, h1), lambda i: (i, 0, 0)),
        compiler_params=_params(),
    )(*args).reshape(n, h1)


def _decode(fp, uv, wxy, w2, b2):
    n, h1 = fp.shape
    s = uv.shape[2]
    fpc = fp.reshape(n, h1, 1)
    return pl.pallas_call(
        _dec_kernel,
        out_shape=jax.ShapeDtypeStruct((n, 3, s), jnp.float32),
        grid=(n,),
        in_specs=[
            pl.BlockSpec((1, h1, 1), lambda i: (i, 0, 0)),
            pl.BlockSpec((1, 2, s), lambda i: (i, 0, 0)),
            _const2(wxy.shape), _const2(w2.shape), _const2(b2.shape),
        ],
        out_specs=pl.BlockSpec((1, 3, s), lambda i: (i, 0, 0)),
        compiler_params=_params(),
    )(fpc, uv, wxy, w2, b2)


def kernel(x, coords, conv1, bn1_g, bn1_b, bn1_m, bn1_v,
           l0_b0_conv1, l0_b0_bn1_g, l0_b0_bn1_b, l0_b0_bn1_m, l0_b0_bn1_v,
           l0_b0_conv2, l0_b0_bn2_g, l0_b0_bn2_b, l0_b0_bn2_m, l0_b0_bn2_v,
           l0_b1_conv1, l0_b1_bn1_g, l0_b1_bn1_b, l0_b1_bn1_m, l0_b1_bn1_v,
           l0_b1_conv2, l0_b1_bn2_g, l0_b1_bn2_b, l0_b1_bn2_m, l0_b1_bn2_v,
           l1_b0_conv1, l1_b0_bn1_g, l1_b0_bn1_b, l1_b0_bn1_m, l1_b0_bn1_v,
           l1_b0_conv2, l1_b0_bn2_g, l1_b0_bn2_b, l1_b0_bn2_m, l1_b0_bn2_v,
           l1_b0_dconv, l1_b0_dbn_g, l1_b0_dbn_b, l1_b0_dbn_m, l1_b0_dbn_v,
           l1_b1_conv1, l1_b1_bn1_g, l1_b1_bn1_b, l1_b1_bn1_m, l1_b1_bn1_v,
           l1_b1_conv2, l1_b1_bn2_g, l1_b1_bn2_b, l1_b1_bn2_m, l1_b1_bn2_v,
           l2_b0_conv1, l2_b0_bn1_g, l2_b0_bn1_b, l2_b0_bn1_m, l2_b0_bn1_v,
           l2_b0_conv2, l2_b0_bn2_g, l2_b0_bn2_b, l2_b0_bn2_m, l2_b0_bn2_v,
           l2_b0_dconv, l2_b0_dbn_g, l2_b0_dbn_b, l2_b0_dbn_m, l2_b0_dbn_v,
           l2_b1_conv1, l2_b1_bn1_g, l2_b1_bn1_b, l2_b1_bn1_m, l2_b1_bn1_v,
           l2_b1_conv2, l2_b1_bn2_g, l2_b1_bn2_b, l2_b1_bn2_m, l2_b1_bn2_v,
           l3_b0_conv1, l3_b0_bn1_g, l3_b0_bn1_b, l3_b0_bn1_m, l3_b0_bn1_v,
           l3_b0_conv2, l3_b0_bn2_g, l3_b0_bn2_b, l3_b0_bn2_m, l3_b0_bn2_v,
           l3_b0_dconv, l3_b0_dbn_g, l3_b0_dbn_b, l3_b0_dbn_m, l3_b0_dbn_v,
           l3_b1_conv1, l3_b1_bn1_g, l3_b1_bn1_b, l3_b1_bn1_m, l3_b1_bn1_v,
           l3_b1_conv2, l3_b1_bn2_g, l3_b1_bn2_b, l3_b1_bn2_m, l3_b1_bn2_v,
           fc1_w, fc1_b, dec1_w, dec1_b, dec2_w, dec2_b):
    xh = jnp.transpose(x, (0, 2, 3, 1)).astype(jnp.bfloat16)
    xs = _s2d(xh)

    pooled = _stem(xs, _w_s2(conv1, 4, _TAP7),
                   *_fold_bn(bn1_g, bn1_b, bn1_m, bn1_v))

    l1 = _layer1(pooled, [
        (_w_s1(l0_b0_conv1), *_fold_bn(l0_b0_bn1_g, l0_b0_bn1_b,
                                       l0_b0_bn1_m, l0_b0_bn1_v)),
        (_w_s1(l0_b0_conv2), *_fold_bn(l0_b0_bn2_g, l0_b0_bn2_b,
                                       l0_b0_bn2_m, l0_b0_bn2_v)),
        (_w_s1(l0_b1_conv1), *_fold_bn(l0_b1_bn1_g, l0_b1_bn1_b,
                                       l0_b1_bn1_m, l0_b1_bn1_v)),
        (_w_s1(l0_b1_conv2), *_fold_bn(l0_b1_bn2_g, l0_b1_bn2_b,
                                       l0_b1_bn2_m, l0_b1_bn2_v)),
    ])

    def layer_args(c1, bn1, dc, dbn, c2, bn2, c3, bn3, c4, bn4):
        return [
            (_w_s2(c1, 2, _TAP3), *_fold_bn(*bn1)),
            (dc[:, :, 0, 0].T.astype(jnp.bfloat16), *_fold_bn(*dbn)),
            (_w_s1(c2), *_fold_bn(*bn2)),
            (_w_s1(c3), *_fold_bn(*bn3)),
            (_w_s1(c4), *_fold_bn(*bn4)),
        ]

    l2 = _layer(_s2d(l1), layer_args(
        l1_b0_conv1, (l1_b0_bn1_g, l1_b0_bn1_b, l1_b0_bn1_m, l1_b0_bn1_v),
        l1_b0_dconv, (l1_b0_dbn_g, l1_b0_dbn_b, l1_b0_dbn_m, l1_b0_dbn_v),
        l1_b0_conv2, (l1_b0_bn2_g, l1_b0_bn2_b, l1_b0_bn2_m, l1_b0_bn2_v),
        l1_b1_conv1, (l1_b1_bn1_g, l1_b1_bn1_b, l1_b1_bn1_m, l1_b1_bn1_v),
        l1_b1_conv2, (l1_b1_bn2_g, l1_b1_bn2_b, l1_b1_bn2_m, l1_b1_bn2_v)),
        64, 128)

    l3 = _layer(_s2d(l2), layer_args(
        l2_b0_conv1, (l2_b0_bn1_g, l2_b0_bn1_b, l2_b0_bn1_m, l2_b0_bn1_v),
        l2_b0_dconv, (l2_b0_dbn_g, l2_b0_dbn_b, l2_b0_dbn_m, l2_b0_dbn_v),
        l2_b0_conv2, (l2_b0_bn2_g, l2_b0_bn2_b, l2_b0_bn2_m, l2_b0_bn2_v),
        l2_b1_conv1, (l2_b1_bn1_g, l2_b1_bn1_b, l2_b1_bn1_m, l2_b1_bn1_v),
        l2_b1_conv2, (l2_b1_bn2_g, l2_b1_bn2_b, l2_b1_bn2_m, l2_b1_bn2_v)),
        128, 256)

    fp = _layer(_s2d(l3), layer_args(
        l3_b0_conv1, (l3_b0_bn1_g, l3_b0_bn1_b, l3_b0_bn1_m, l3_b0_bn1_v),
        l3_b0_dconv, (l3_b0_dbn_g, l3_b0_dbn_b, l3_b0_dbn_m, l3_b0_dbn_v),
        l3_b0_conv2, (l3_b0_bn2_g, l3_b0_bn2_b, l3_b0_bn2_m, l3_b0_bn2_v),
        l3_b1_conv1, (l3_b1_bn1_g, l3_b1_bn1_b, l3_b1_bn1_m, l3_b1_bn1_v),
        l3_b1_conv2, (l3_b1_bn2_g, l3_b1_bn2_b, l3_b1_bn2_m, l3_b1_bn2_v)),
        256, 512,
        head=(fc1_w.T.astype(jnp.bfloat16),
              fc1_b.reshape(1, -1).astype(jnp.float32),
              dec1_w[:, :256, 0].T.astype(jnp.bfloat16),
              dec1_b.reshape(1, -1).astype(jnp.float32)))

    uv = jnp.transpose(coords, (0, 2, 1)).astype(jnp.float32)
    return _decode(fp, uv,
                   dec1_w[:, 256:258, 0].astype(jnp.float32),
                   dec2_w[:, :, 0].astype(jnp.float32),
                   dec2_b.reshape(-1, 1).astype(jnp.float32))
